# baseline (speedup 1.0000x reference)
import functools

import jax
import jax.numpy as jnp
from jax import lax
from jax.experimental import pallas as pl
from jax.experimental.pallas import tpu as pltpu
from jax.experimental.pallas import tpu_sc as plsc

N = 320000
D = 128
NSEG = 512
ACC_W = D + 16
NC, NS, L = 2, 16, 16
NW = NC * NS
ROWS_PER_W = N // NW
RBLK = 80
NBLK = ROWS_PER_W // RBLK
GBLK = 2500


def _tc_gates_body(h_ref, wg_ref, bg_ref, o_ref):
  x = jnp.sum(h_ref[...] * wg_ref[...][None, None, :], axis=2)
  o_ref[...] = jax.nn.sigmoid(x + bg_ref[0, 0])


def _tc_gates(h, wg, bg):
  nrow = N // GBLK
  sub = 8
  h3 = h.reshape(nrow, GBLK, D)
  return pl.pallas_call(
      _tc_gates_body,
      grid=(nrow // sub,),
      in_specs=[
          pl.BlockSpec((sub, GBLK, D), lambda i: (i, 0, 0)),
          pl.BlockSpec((D,), lambda i: (0,)),
          pl.BlockSpec((1, 1), lambda i: (0, 0), memory_space=pltpu.SMEM),
      ],
      out_specs=pl.BlockSpec((sub, GBLK), lambda i: (i, 0)),
      out_shape=jax.ShapeDtypeStruct((nrow, GBLK), jnp.float32),
  )(h3, wg, bg)


def _sc_segsum(h, idx, gates, zacc):
  mesh = plsc.VectorSubcoreMesh(
      core_axis_name="c", subcore_axis_name="s", num_cores=NC,
      num_subcores=NS)

  @functools.partial(
      pl.kernel,
      out_type=jax.ShapeDtypeStruct((NW, NSEG, ACC_W), jnp.float32),
      mesh=mesh,
      scratch_types=[
          pltpu.VMEM((NSEG, ACC_W), jnp.float32),
          pltpu.VMEM((RBLK, D), jnp.float32),
          pltpu.VMEM((RBLK, D), jnp.float32),
          pltpu.VMEM((RBLK,), jnp.int32),
          pltpu.VMEM((RBLK,), jnp.int32),
          pltpu.VMEM((RBLK,), jnp.float32),
          pltpu.VMEM((RBLK,), jnp.float32),
          pltpu.SemaphoreType.DMA,
          pltpu.SemaphoreType.DMA,
      ],
      compiler_params=pltpu.CompilerParams(needs_layout_passes=False,
                                           use_tc_tiling_on_sc=False),
  )
  def k(h_hbm, idx_hbm, g_hbm, z_hbm, out_hbm,
        acc_v, h_v0, h_v1, idx_v0, idx_v1, g_v0, g_v1, sem0, sem1):
    wid = lax.axis_index("c") * NS + lax.axis_index("s")
    base0 = wid * ROWS_PER_W
    bufs = ((h_v0, idx_v0, g_v0, sem0), (h_v1, idx_v1, g_v1, sem1))

    def start_copies(b, hbuf, ibuf, gbuf, sem):
      base = base0 + b * RBLK
      pltpu.make_async_copy(h_hbm.at[pl.ds(base, RBLK), :], hbuf, sem).start()
      pltpu.make_async_copy(idx_hbm.at[pl.ds(base, RBLK)], ibuf, sem).start()
      pltpu.make_async_copy(g_hbm.at[pl.ds(base, RBLK)], gbuf, sem).start()

    def wait_copies(b, hbuf, ibuf, gbuf, sem):
      base = base0 + b * RBLK
      pltpu.make_async_copy(h_hbm.at[pl.ds(base, RBLK), :], hbuf, sem).wait()
      pltpu.make_async_copy(idx_hbm.at[pl.ds(base, RBLK)], ibuf, sem).wait()
      pltpu.make_async_copy(g_hbm.at[pl.ds(base, RBLK)], gbuf, sem).wait()

    pltpu.sync_copy(z_hbm, acc_v)
    NK = D // L

    def process_block(h_v, idx_v, g_v):
      def group_body(g, _):
        iv = idx_v[pl.ds(g * L, L)]
        gv = g_v[pl.ds(g * L, L)]
        gb = [jnp.broadcast_to(gv[j], (L,)) for j in range(L)]
        gsum = jnp.broadcast_to(jnp.sum(gv), (L,))
        seg0 = iv[0]
        seg15 = iv[L - 1]

        def fast(_):
          a = [gb[0] * h_v[g * L, pl.ds(16 * k2, 16)] for k2 in range(NK)]
          for j in range(1, L):
            for k2 in range(NK):
              a[k2] = a[k2] + gb[j] * h_v[g * L + j, pl.ds(16 * k2, 16)]
          for k2 in range(NK):
            plsc.addupdate(acc_v.at[seg0, pl.ds(16 * k2, 16)], a[k2])
          plsc.addupdate(acc_v.at[seg0, pl.ds(D, 16)], gsum)
          return 0

        def slow(_):
          for j in range(L):
            seg = iv[j]
            for k2 in range(NK):
              plsc.addupdate(acc_v.at[seg, pl.ds(16 * k2, 16)],
                             gb[j] * h_v[g * L + j, pl.ds(16 * k2, 16)])
            plsc.addupdate(acc_v.at[seg, pl.ds(D, 16)], gb[j])
          return 0

        lax.cond(seg0 == seg15, fast, slow, 0)
        return 0

      lax.fori_loop(0, RBLK // L, group_body, 0)

    start_copies(0, *bufs[0])
    start_copies(1, *bufs[1])

    def pair_body(p, _):
      for par in range(2):
        b = 2 * p + par
        hbuf, ibuf, gbuf, sem = bufs[par]

        @pl.when(b < NBLK)
        def _():
          wait_copies(b, hbuf, ibuf, gbuf, sem)
          process_block(hbuf, ibuf, gbuf)

          @pl.when(b + 2 < NBLK)
          def _():
            start_copies(b + 2, hbuf, ibuf, gbuf, sem)
      return 0

    lax.fori_loop(0, (NBLK + 1) // 2, pair_body, 0)
    pltpu.sync_copy(acc_v, out_hbm.at[wid])

  return k(h, idx, gates, zacc)


def _tc_finish_body(p_ref, wt_ref, bt_ref, wo_ref, bo_ref, o_ref):
  a = jnp.sum(p_ref[...], axis=0)
  g = a[:, :D]
  c = a[:, D:D + 1]
  pooled = jnp.dot(g, wt_ref[...], preferred_element_type=jnp.float32)
  pooled = pooled + c * bt_ref[...][None, :]
  out = jnp.dot(pooled, wo_ref[...], preferred_element_type=jnp.float32)
  o_ref[...] = out + bo_ref[...][None, :]


def kernel(h, batch_idx, W_gate, b_gate, W_t, b_t, W_out, b_out):
  idx = batch_idx.astype(jnp.int32)
  zacc = jnp.zeros((NSEG, ACC_W), jnp.float32)

  gates = _tc_gates(h, W_gate.reshape(D), b_gate.reshape(1, 1)).reshape(N)
  partials = _sc_segsum(h, idx, gates, zacc)

  return pl.pallas_call(
      _tc_finish_body,
      out_shape=jax.ShapeDtypeStruct((NSEG, D), jnp.float32),
  )(partials, W_t, b_t, W_out, b_out)


# speedup vs baseline: 1.3514x; 1.3514x over previous
import functools

import jax
import jax.numpy as jnp
from jax import lax
from jax.experimental import pallas as pl
from jax.experimental.pallas import tpu as pltpu
from jax.experimental.pallas import tpu_sc as plsc

N = 320000
D = 128
NSEG = 512
ACC_W = D + 16
NC, NS, L = 2, 16, 16
NW = NC * NS
ROWS_PER_W = N // NW
RBLK = 80
NBLK = ROWS_PER_W // RBLK
GBLK = 2000


def _tc_gates_body(h_ref, wg_ref, bg_ref, o_ref):
  x = jnp.sum(h_ref[...] * wg_ref[...][None, None, :], axis=2)
  o_ref[...] = jax.nn.sigmoid(x + bg_ref[0, 0])


def _tc_gates(h3, wg, bg, c0, nchunks):
  sub = 8
  b0 = c0 // sub
  return pl.pallas_call(
      _tc_gates_body,
      grid=(nchunks // sub,),
      in_specs=[
          pl.BlockSpec((sub, GBLK, D), lambda i: (b0 + i, 0, 0)),
          pl.BlockSpec((D,), lambda i: (0,)),
          pl.BlockSpec((1, 1), lambda i: (0, 0), memory_space=pltpu.SMEM),
      ],
      out_specs=pl.BlockSpec((sub, GBLK), lambda i: (i, 0)),
      out_shape=jax.ShapeDtypeStruct((nchunks, GBLK), jnp.float32),
  )(h3, wg, bg)


def _sc_segsum(h, idx, gates, zacc, row0, rows_per_w):
  nblk = rows_per_w // RBLK
  mesh = plsc.VectorSubcoreMesh(
      core_axis_name="c", subcore_axis_name="s", num_cores=NC,
      num_subcores=NS)

  @functools.partial(
      pl.kernel,
      out_type=jax.ShapeDtypeStruct((NW, NSEG, ACC_W), jnp.float32),
      mesh=mesh,
      scratch_types=[
          pltpu.VMEM((NSEG, ACC_W), jnp.float32),
          pltpu.VMEM((RBLK, D), jnp.float32),
          pltpu.VMEM((RBLK, D), jnp.float32),
          pltpu.VMEM((RBLK,), jnp.int32),
          pltpu.VMEM((RBLK,), jnp.int32),
          pltpu.VMEM((RBLK,), jnp.float32),
          pltpu.VMEM((RBLK,), jnp.float32),
          pltpu.SemaphoreType.DMA,
          pltpu.SemaphoreType.DMA,
      ],
      compiler_params=pltpu.CompilerParams(needs_layout_passes=False,
                                           use_tc_tiling_on_sc=False),
  )
  def k(h_hbm, idx_hbm, g_hbm, z_hbm, out_hbm,
        acc_v, h_v0, h_v1, idx_v0, idx_v1, g_v0, g_v1, sem0, sem1):
    wid = lax.axis_index("c") * NS + lax.axis_index("s")
    lbase0 = wid * rows_per_w
    bufs = ((h_v0, idx_v0, g_v0, sem0), (h_v1, idx_v1, g_v1, sem1))

    def start_copies(b, hbuf, ibuf, gbuf, sem):
      lbase = lbase0 + b * RBLK
      base = row0 + lbase
      pltpu.make_async_copy(h_hbm.at[pl.ds(base, RBLK), :], hbuf, sem).start()
      pltpu.make_async_copy(idx_hbm.at[pl.ds(base, RBLK)], ibuf, sem).start()
      pltpu.make_async_copy(g_hbm.at[pl.ds(lbase, RBLK)], gbuf, sem).start()

    def wait_copies(b, hbuf, ibuf, gbuf, sem):
      lbase = lbase0 + b * RBLK
      base = row0 + lbase
      pltpu.make_async_copy(h_hbm.at[pl.ds(base, RBLK), :], hbuf, sem).wait()
      pltpu.make_async_copy(idx_hbm.at[pl.ds(base, RBLK)], ibuf, sem).wait()
      pltpu.make_async_copy(g_hbm.at[pl.ds(lbase, RBLK)], gbuf, sem).wait()

    pltpu.sync_copy(z_hbm, acc_v)
    NK = D // L

    def process_block(h_v, idx_v, g_v):
      def group_body(g, _):
        iv = idx_v[pl.ds(g * L, L)]
        gv = g_v[pl.ds(g * L, L)]
        gb = [jnp.broadcast_to(gv[j], (L,)) for j in range(L)]
        gsum = jnp.broadcast_to(jnp.sum(gv), (L,))
        seg0 = iv[0]
        seg15 = iv[L - 1]

        def fast(_):
          a = [gb[0] * h_v[g * L, pl.ds(16 * k2, 16)] for k2 in range(NK)]
          for j in range(1, L):
            for k2 in range(NK):
              a[k2] = a[k2] + gb[j] * h_v[g * L + j, pl.ds(16 * k2, 16)]
          for k2 in range(NK):
            plsc.addupdate(acc_v.at[seg0, pl.ds(16 * k2, 16)], a[k2])
          plsc.addupdate(acc_v.at[seg0, pl.ds(D, 16)], gsum)
          return 0

        def slow(_):
          for j in range(L):
            seg = iv[j]
            for k2 in range(NK):
              plsc.addupdate(acc_v.at[seg, pl.ds(16 * k2, 16)],
                             gb[j] * h_v[g * L + j, pl.ds(16 * k2, 16)])
            plsc.addupdate(acc_v.at[seg, pl.ds(D, 16)], gb[j])
          return 0

        lax.cond(seg0 == seg15, fast, slow, 0)
        return 0

      lax.fori_loop(0, RBLK // L, group_body, 0)

    start_copies(0, *bufs[0])
    start_copies(1, *bufs[1])

    def pair_body(p, _):
      for par in range(2):
        b = 2 * p + par
        hbuf, ibuf, gbuf, sem = bufs[par]

        @pl.when(b < nblk)
        def _():
          wait_copies(b, hbuf, ibuf, gbuf, sem)
          process_block(hbuf, ibuf, gbuf)

          @pl.when(b + 2 < nblk)
          def _():
            start_copies(b + 2, hbuf, ibuf, gbuf, sem)
      return 0

    lax.fori_loop(0, (nblk + 1) // 2, pair_body, 0)
    pltpu.sync_copy(acc_v, out_hbm.at[wid])

  return k(h, idx, gates, zacc)


def _tc_finish_body(p_ref, q_ref, wt_ref, bt_ref, wo_ref, bo_ref, o_ref):
  a = jnp.sum(p_ref[...], axis=0) + jnp.sum(q_ref[...], axis=0)
  g = a[:, :D]
  c = a[:, D:D + 1]
  pooled = jnp.dot(g, wt_ref[...], preferred_element_type=jnp.float32)
  pooled = pooled + c * bt_ref[...][None, :]
  out = jnp.dot(pooled, wo_ref[...], preferred_element_type=jnp.float32)
  o_ref[...] = out + bo_ref[...][None, :]


SPLIT = 128000


def kernel(h, batch_idx, W_gate, b_gate, W_t, b_t, W_out, b_out):
  idx = batch_idx.astype(jnp.int32)
  zacc = jnp.zeros((NSEG, ACC_W), jnp.float32)
  h3 = h.reshape(N // GBLK, GBLK, D)
  wg = W_gate.reshape(D)
  bg = b_gate.reshape(1, 1)

  ca, cb = SPLIT // GBLK, (N - SPLIT) // GBLK
  gates_a = _tc_gates(h3, wg, bg, 0, ca).reshape(SPLIT)
  p_a = _sc_segsum(h, idx, gates_a, zacc, 0, SPLIT // NW)
  gates_b = _tc_gates(h3, wg, bg, ca, cb).reshape(N - SPLIT)
  p_b = _sc_segsum(h, idx, gates_b, zacc, SPLIT, (N - SPLIT) // NW)

  return pl.pallas_call(
      _tc_finish_body,
      out_shape=jax.ShapeDtypeStruct((NSEG, D), jnp.float32),
  )(p_a, p_b, W_t, b_t, W_out, b_out)


# speedup vs baseline: 1.4753x; 1.0917x over previous
import functools

import jax
import jax.numpy as jnp
from jax import lax
from jax.experimental import pallas as pl
from jax.experimental.pallas import tpu as pltpu
from jax.experimental.pallas import tpu_sc as plsc

N = 320000
D = 128
NSEG = 512
ACC_W = D + 16
NC, NS, L = 2, 16, 16
NW = NC * NS
ROWS_PER_W = N // NW
RBLK = 80
NBLK = ROWS_PER_W // RBLK
GBLK = 2000


def _tc_gates_body(h_ref, wg_ref, bg_ref, o_ref):
  x = jnp.sum(h_ref[...] * wg_ref[...][None, None, :], axis=2)
  o_ref[...] = jax.nn.sigmoid(x + bg_ref[0, 0])


def _tc_gates(h3, wg, bg, c0, nchunks):
  sub = 8
  b0 = c0 // sub
  return pl.pallas_call(
      _tc_gates_body,
      grid=(nchunks // sub,),
      in_specs=[
          pl.BlockSpec((sub, GBLK, D), lambda i: (b0 + i, 0, 0)),
          pl.BlockSpec((D,), lambda i: (0,)),
          pl.BlockSpec((1, 1), lambda i: (0, 0), memory_space=pltpu.SMEM),
      ],
      out_specs=pl.BlockSpec((sub, GBLK), lambda i: (i, 0)),
      out_shape=jax.ShapeDtypeStruct((nchunks, GBLK), jnp.float32),
  )(h3, wg, bg)


def _sc_segsum(h, idx, gates, zacc, row0, rows_per_w):
  nblk = rows_per_w // RBLK
  mesh = plsc.VectorSubcoreMesh(
      core_axis_name="c", subcore_axis_name="s", num_cores=NC,
      num_subcores=NS)

  @functools.partial(
      pl.kernel,
      out_type=jax.ShapeDtypeStruct((NW, NSEG, ACC_W), jnp.float32),
      mesh=mesh,
      scratch_types=[
          pltpu.VMEM((NSEG, ACC_W), jnp.float32),
          pltpu.VMEM((RBLK, D), jnp.float32),
          pltpu.VMEM((RBLK, D), jnp.float32),
          pltpu.VMEM((RBLK,), jnp.int32),
          pltpu.VMEM((RBLK,), jnp.int32),
          pltpu.VMEM((RBLK,), jnp.float32),
          pltpu.VMEM((RBLK,), jnp.float32),
          pltpu.SemaphoreType.DMA,
          pltpu.SemaphoreType.DMA,
          pltpu.SemaphoreType.DMA,
      ],
      compiler_params=pltpu.CompilerParams(needs_layout_passes=False,
                                           use_tc_tiling_on_sc=False),
  )
  def k(h_hbm, idx_hbm, g_hbm, z_hbm, out_hbm,
        acc_v, h_v0, h_v1, idx_v0, idx_v1, g_v0, g_v1, sem0, sem1, zsem):
    wid = lax.axis_index("c") * NS + lax.axis_index("s")
    lbase0 = wid * rows_per_w
    bufs = ((h_v0, idx_v0, g_v0, sem0), (h_v1, idx_v1, g_v1, sem1))

    def start_copies(b, hbuf, ibuf, gbuf, sem):
      lbase = lbase0 + b * RBLK
      base = row0 + lbase
      pltpu.make_async_copy(h_hbm.at[pl.ds(base, RBLK), :], hbuf, sem).start()
      pltpu.make_async_copy(idx_hbm.at[pl.ds(base, RBLK)], ibuf, sem).start()
      pltpu.make_async_copy(g_hbm.at[pl.ds(lbase, RBLK)], gbuf, sem).start()

    def wait_copies(b, hbuf, ibuf, gbuf, sem):
      lbase = lbase0 + b * RBLK
      base = row0 + lbase
      pltpu.make_async_copy(h_hbm.at[pl.ds(base, RBLK), :], hbuf, sem).wait()
      pltpu.make_async_copy(idx_hbm.at[pl.ds(base, RBLK)], ibuf, sem).wait()
      pltpu.make_async_copy(g_hbm.at[pl.ds(lbase, RBLK)], gbuf, sem).wait()

    zcopy = pltpu.make_async_copy(z_hbm, acc_v, zsem)
    zcopy.start()
    NK = D // L

    def process_block(h_v, idx_v, g_v):
      def group_body(g, _):
        iv = idx_v[pl.ds(g * L, L)]
        gv = g_v[pl.ds(g * L, L)]
        gb = [jnp.broadcast_to(gv[j], (L,)) for j in range(L)]
        gsum = jnp.broadcast_to(jnp.sum(gv), (L,))
        seg0 = iv[0]
        seg15 = iv[L - 1]

        def fast(_):
          a = [gb[0] * h_v[g * L, pl.ds(16 * k2, 16)] for k2 in range(NK)]
          for j in range(1, L):
            for k2 in range(NK):
              a[k2] = a[k2] + gb[j] * h_v[g * L + j, pl.ds(16 * k2, 16)]
          for k2 in range(NK):
            plsc.addupdate(acc_v.at[seg0, pl.ds(16 * k2, 16)], a[k2])
          plsc.addupdate(acc_v.at[seg0, pl.ds(D, 16)], gsum)
          return 0

        def slow(_):
          for j in range(L):
            seg = iv[j]
            for k2 in range(NK):
              plsc.addupdate(acc_v.at[seg, pl.ds(16 * k2, 16)],
                             gb[j] * h_v[g * L + j, pl.ds(16 * k2, 16)])
            plsc.addupdate(acc_v.at[seg, pl.ds(D, 16)], gb[j])
          return 0

        lax.cond(seg0 == seg15, fast, slow, 0)
        return 0

      lax.fori_loop(0, RBLK // L, group_body, 0)

    start_copies(0, *bufs[0])
    start_copies(1, *bufs[1])
    zcopy.wait()

    def pair_body(p, _):
      for par in range(2):
        b = 2 * p + par
        hbuf, ibuf, gbuf, sem = bufs[par]

        @pl.when(b < nblk)
        def _():
          wait_copies(b, hbuf, ibuf, gbuf, sem)
          process_block(hbuf, ibuf, gbuf)

          @pl.when(b + 2 < nblk)
          def _():
            start_copies(b + 2, hbuf, ibuf, gbuf, sem)
      return 0

    lax.fori_loop(0, (nblk + 1) // 2, pair_body, 0)
    pltpu.sync_copy(acc_v, out_hbm.at[wid])

  return k(h, idx, gates, zacc)


def _tc_finish_body(p_ref, wt_ref, bt_ref, wo_ref, bo_ref, o_ref):
  a = jnp.sum(p_ref[...], axis=0)
  g = a[:, :D]
  c = a[:, D:D + 1]
  pooled = jnp.dot(g, wt_ref[...], preferred_element_type=jnp.float32)
  pooled = pooled + c * bt_ref[...][None, :]
  out = jnp.dot(pooled, wo_ref[...], preferred_element_type=jnp.float32)
  o_ref[...] = out + bo_ref[...][None, :]


SPLIT = 128000


def kernel(h, batch_idx, W_gate, b_gate, W_t, b_t, W_out, b_out):
  idx = batch_idx.astype(jnp.int32)
  zacc = jnp.zeros((NSEG, ACC_W), jnp.float32)
  h3 = h.reshape(N // GBLK, GBLK, D)
  wg = W_gate.reshape(D)
  bg = b_gate.reshape(1, 1)

  gates = _tc_gates(h3, wg, bg, 0, N // GBLK).reshape(N)
  partials = _sc_segsum(h, idx, gates, zacc, 0, ROWS_PER_W)

  return pl.pallas_call(
      _tc_finish_body,
      out_shape=jax.ShapeDtypeStruct((NSEG, D), jnp.float32),
  )(partials, W_t, b_t, W_out, b_out)


# speedup vs baseline: 1.5007x; 1.0172x over previous
import functools

import jax
import jax.numpy as jnp
from jax import lax
from jax.experimental import pallas as pl
from jax.experimental.pallas import tpu as pltpu
from jax.experimental.pallas import tpu_sc as plsc

N = 320000
D = 128
NSEG = 512
ACC_W = D + 16
NC, NS, L = 2, 16, 16
NW = NC * NS
ROWS_PER_W = N // NW
RBLK = 80
NBLK = ROWS_PER_W // RBLK
GBLK = 2000


def _tc_gates_body(h_ref, wg_ref, bg_ref, o_ref):
  x = jnp.sum(h_ref[...] * wg_ref[...][None, None, :], axis=2)
  o_ref[...] = jax.nn.sigmoid(x + bg_ref[0, 0])


def _tc_gates(h3, wg, bg, c0, nchunks):
  sub = 16
  b0 = c0 // sub
  return pl.pallas_call(
      _tc_gates_body,
      grid=(nchunks // sub,),
      in_specs=[
          pl.BlockSpec((sub, GBLK, D), lambda i: (b0 + i, 0, 0)),
          pl.BlockSpec((D,), lambda i: (0,)),
          pl.BlockSpec((1, 1), lambda i: (0, 0), memory_space=pltpu.SMEM),
      ],
      out_specs=pl.BlockSpec((sub, GBLK), lambda i: (i, 0)),
      out_shape=jax.ShapeDtypeStruct((nchunks, GBLK), jnp.float32),
  )(h3, wg, bg)


def _sc_segsum(h, idx, gates, zacc, row0, rows_per_w):
  nblk = rows_per_w // RBLK
  mesh = plsc.VectorSubcoreMesh(
      core_axis_name="c", subcore_axis_name="s", num_cores=NC,
      num_subcores=NS)

  @functools.partial(
      pl.kernel,
      out_type=jax.ShapeDtypeStruct((NW, NSEG, ACC_W), jnp.float32),
      mesh=mesh,
      scratch_types=[
          pltpu.VMEM((NSEG, ACC_W), jnp.float32),
          pltpu.VMEM((RBLK, D), jnp.float32),
          pltpu.VMEM((RBLK, D), jnp.float32),
          pltpu.VMEM((RBLK,), jnp.int32),
          pltpu.VMEM((RBLK,), jnp.int32),
          pltpu.VMEM((RBLK,), jnp.float32),
          pltpu.VMEM((RBLK,), jnp.float32),
          pltpu.SemaphoreType.DMA,
          pltpu.SemaphoreType.DMA,
          pltpu.SemaphoreType.DMA,
      ],
      compiler_params=pltpu.CompilerParams(needs_layout_passes=False,
                                           use_tc_tiling_on_sc=False),
  )
  def k(h_hbm, idx_hbm, g_hbm, z_hbm, out_hbm,
        acc_v, h_v0, h_v1, idx_v0, idx_v1, g_v0, g_v1, sem0, sem1, zsem):
    wid = lax.axis_index("c") * NS + lax.axis_index("s")
    lbase0 = wid * rows_per_w
    bufs = ((h_v0, idx_v0, g_v0, sem0), (h_v1, idx_v1, g_v1, sem1))

    def start_copies(b, hbuf, ibuf, gbuf, sem):
      lbase = lbase0 + b * RBLK
      base = row0 + lbase
      pltpu.make_async_copy(h_hbm.at[pl.ds(base, RBLK), :], hbuf, sem).start()
      pltpu.make_async_copy(idx_hbm.at[pl.ds(base, RBLK)], ibuf, sem).start()
      pltpu.make_async_copy(g_hbm.at[pl.ds(lbase, RBLK)], gbuf, sem).start()

    def wait_copies(b, hbuf, ibuf, gbuf, sem):
      lbase = lbase0 + b * RBLK
      base = row0 + lbase
      pltpu.make_async_copy(h_hbm.at[pl.ds(base, RBLK), :], hbuf, sem).wait()
      pltpu.make_async_copy(idx_hbm.at[pl.ds(base, RBLK)], ibuf, sem).wait()
      pltpu.make_async_copy(g_hbm.at[pl.ds(lbase, RBLK)], gbuf, sem).wait()

    zcopy = pltpu.make_async_copy(z_hbm, acc_v, zsem)
    zcopy.start()
    NK = D // L

    def process_block(h_v, idx_v, g_v):
      def group_body(g, _):
        iv = idx_v[pl.ds(g * L, L)]
        gv = g_v[pl.ds(g * L, L)]
        gb = [jnp.broadcast_to(gv[j], (L,)) for j in range(L)]
        gsum = jnp.broadcast_to(jnp.sum(gv), (L,))
        seg0 = iv[0]
        seg15 = iv[L - 1]

        def fast(_):
          a = [gb[0] * h_v[g * L, pl.ds(16 * k2, 16)] for k2 in range(NK)]
          for j in range(1, L):
            for k2 in range(NK):
              a[k2] = a[k2] + gb[j] * h_v[g * L + j, pl.ds(16 * k2, 16)]
          for k2 in range(NK):
            plsc.addupdate(acc_v.at[seg0, pl.ds(16 * k2, 16)], a[k2])
          plsc.addupdate(acc_v.at[seg0, pl.ds(D, 16)], gsum)
          return 0

        def slow(_):
          for j in range(L):
            seg = iv[j]
            for k2 in range(NK):
              plsc.addupdate(acc_v.at[seg, pl.ds(16 * k2, 16)],
                             gb[j] * h_v[g * L + j, pl.ds(16 * k2, 16)])
            plsc.addupdate(acc_v.at[seg, pl.ds(D, 16)], gb[j])
          return 0

        lax.cond(seg0 == seg15, fast, slow, 0)
        return 0

      lax.fori_loop(0, RBLK // L, group_body, 0)

    start_copies(0, *bufs[0])
    start_copies(1, *bufs[1])
    zcopy.wait()

    def pair_body(p, _):
      for par in range(2):
        b = 2 * p + par
        hbuf, ibuf, gbuf, sem = bufs[par]

        @pl.when(b < nblk)
        def _():
          wait_copies(b, hbuf, ibuf, gbuf, sem)
          process_block(hbuf, ibuf, gbuf)

          @pl.when(b + 2 < nblk)
          def _():
            start_copies(b + 2, hbuf, ibuf, gbuf, sem)
      return 0

    lax.fori_loop(0, (nblk + 1) // 2, pair_body, 0)
    pltpu.sync_copy(acc_v, out_hbm.at[wid])

  return k(h, idx, gates, zacc)


def _tc_finish_body(p_ref, wt_ref, bt_ref, wo_ref, bo_ref, o_ref):
  a = jnp.sum(p_ref[...], axis=0)
  g = a[:, :D]
  c = a[:, D:D + 1]
  pooled = jnp.dot(g, wt_ref[...], preferred_element_type=jnp.float32)
  pooled = pooled + c * bt_ref[...][None, :]
  out = jnp.dot(pooled, wo_ref[...], preferred_element_type=jnp.float32)
  o_ref[...] = out + bo_ref[...][None, :]


SPLIT = 128000


def kernel(h, batch_idx, W_gate, b_gate, W_t, b_t, W_out, b_out):
  idx = batch_idx.astype(jnp.int32)
  zacc = jnp.zeros((NSEG, ACC_W), jnp.float32)
  h3 = h.reshape(N // GBLK, GBLK, D)
  wg = W_gate.reshape(D)
  bg = b_gate.reshape(1, 1)

  gates = _tc_gates(h3, wg, bg, 0, N // GBLK).reshape(N)
  partials = _sc_segsum(h, idx, gates, zacc, 0, ROWS_PER_W)

  return pl.pallas_call(
      _tc_finish_body,
      out_shape=jax.ShapeDtypeStruct((NSEG, D), jnp.float32),
  )(partials, W_t, b_t, W_out, b_out)
